# baseline (speedup 1.0000x reference)
import functools

import jax
import jax.numpy as jnp
from jax import lax
from jax.experimental import pallas as pl
from jax.experimental.pallas import tpu as pltpu
from jax.experimental.pallas import tpu_sc as plsc

D = 1024
E = 8
H = 512
T = 2048
BT = 256
NBLK = T // BT
BG = 128
NP = 4096 + E * BG
NB = NP // BG
NBE = 64
NW = 32
SLOTS = NP // NW
GCH = 32
TOK = T // NW
CCH = 32


def _dot_t(a, b):
    return lax.dot_general(a, b, (((1,), (1,)), ((), ())),
                           preferred_element_type=jnp.float32)


def _router_body(x_ref, wr_ref, pos0_ref, pos1_ref, w0_ref, w1_ref, be_ref,
                 acc_ref, base_ref, poffs_ref):
    p = pl.program_id(0)
    b = pl.program_id(1)
    x = x_ref[...]
    logits = _dot_t(x, wr_ref[...])
    iota_e = lax.broadcasted_iota(jnp.int32, (BT, E), 1)
    m0 = jnp.max(logits, axis=1, keepdims=True)
    a0 = jnp.min(jnp.where(logits >= m0, iota_e, E), axis=1, keepdims=True)
    oh0 = iota_e == a0
    masked = jnp.where(oh0, -jnp.inf, logits)
    m1 = jnp.max(masked, axis=1, keepdims=True)
    a1 = jnp.min(jnp.where(masked >= m1, iota_e, E), axis=1, keepdims=True)
    oh1 = iota_e == a1
    oh0f = oh0.astype(jnp.float32)
    oh1f = oh1.astype(jnp.float32)
    c01 = oh0f + oh1f

    @pl.when(p == 0)
    def _():
        @pl.when(b == 0)
        def _():
            acc_ref[...] = jnp.zeros((1, E), jnp.float32)
        base_ref[pl.ds(b, 1), :] = acc_ref[...]
        acc_ref[...] = acc_ref[...] + jnp.sum(c01, axis=0, keepdims=True)

    @pl.when((p == 1) & (b == 0))
    def _():
        cnti = acc_ref[...].astype(jnp.int32)
        padded = ((cnti + (BG - 1)) // BG) * BG
        paddedf = padded.astype(jnp.float32)
        r8 = lax.broadcasted_iota(jnp.int32, (E, E), 0)
        c8 = lax.broadcasted_iota(jnp.int32, (E, E), 1)
        ltri = (r8 < c8).astype(jnp.float32)
        poffs = jnp.dot(paddedf, ltri, preferred_element_type=jnp.float32)
        poffs_ref[...] = poffs
        cbf = (poffs + paddedf) * (1.0 / BG)
        bi = lax.broadcasted_iota(jnp.int32, (NBE, E), 0).astype(jnp.float32)
        be = jnp.sum((bi >= cbf[0:1, :]).astype(jnp.float32),
                     axis=1, keepdims=True)
        be_ref[...] = jnp.minimum(be, float(E - 1)).astype(jnp.int32)

    @pl.when(p == 1)
    def _():
        rr = lax.broadcasted_iota(jnp.int32, (BT, BT), 0)
        cc = lax.broadcasted_iota(jnp.int32, (BT, BT), 1)
        strict = (cc < rr).astype(jnp.float32)
        excl = jnp.dot(strict, c01, preferred_element_type=jnp.float32)
        tot = base_ref[pl.ds(b, 1), :] + excl + poffs_ref[...]
        pos0_ref[...] = jnp.sum(oh0f * tot, axis=1,
                                keepdims=True).astype(jnp.int32)
        pos1_ref[...] = jnp.sum(oh1f * tot, axis=1,
                                keepdims=True).astype(jnp.int32)
        dd = jnp.exp(m1 - m0)
        ss = 1.0 + dd
        w0_ref[...] = 1.0 / ss
        w1_ref[...] = dd / ss


def _router(xf, wr):
    return pl.pallas_call(
        _router_body,
        grid=(2, NBLK),
        in_specs=[
            pl.BlockSpec((BT, D), lambda p, b: (b, 0)),
            pl.BlockSpec((E, D), lambda p, b: (0, 0)),
        ],
        out_specs=[
            pl.BlockSpec((BT, 1), lambda p, b: (b, 0)),
            pl.BlockSpec((BT, 1), lambda p, b: (b, 0)),
            pl.BlockSpec((BT, 1), lambda p, b: (b, 0)),
            pl.BlockSpec((BT, 1), lambda p, b: (b, 0)),
            pl.BlockSpec((NBE, 1), lambda p, b: (0, 0)),
        ],
        out_shape=[
            jax.ShapeDtypeStruct((T, 1), jnp.int32),
            jax.ShapeDtypeStruct((T, 1), jnp.int32),
            jax.ShapeDtypeStruct((T, 1), jnp.float32),
            jax.ShapeDtypeStruct((T, 1), jnp.float32),
            jax.ShapeDtypeStruct((NBE, 1), jnp.int32),
        ],
        scratch_shapes=[
            pltpu.VMEM((1, E), jnp.float32),
            pltpu.VMEM((NBLK, E), jnp.float32),
            pltpu.VMEM((1, E), jnp.float32),
        ],
    )(xf, wr)


def _sc_dispatch_body(pos0_hbm, pos1_hbm, w0_hbm, w1_hbm, xf_hbm,
                      xs_hbm, ws_hbm,
                      pos0_v, pos1_v, w0_v, w1_v, perm_v, wsort_v,
                      buf_v, sem):
    wid = lax.axis_index("s") * 2 + lax.axis_index("c")
    base = wid * SLOTS
    pltpu.sync_copy(pos0_hbm, pos0_v)
    pltpu.sync_copy(pos1_hbm, pos1_v)
    pltpu.sync_copy(w0_hbm, w0_v)
    pltpu.sync_copy(w1_hbm, w1_v)
    zi = jnp.zeros((16,), jnp.int32)
    zf = jnp.zeros((16,), jnp.float32)
    for i in range(SLOTS // 16):
        perm_v[pl.ds(i * 16, 16)] = zi
        wsort_v[pl.ds(i * 16, 16)] = zf
    lane = lax.iota(jnp.int32, 16)

    def scan(pos_ref, w_ref):
        def body(c, carry):
            off = c * 16
            pv = pos_ref[pl.ds(off, 16)]
            wv = w_ref[pl.ds(off, 16)]
            tv = lane + off
            msk = (pv >= base) & (pv < base + SLOTS)
            rel = jnp.where(msk, pv - base, 0)
            plsc.store_scatter(perm_v, [rel], tv, mask=msk)
            plsc.store_scatter(wsort_v, [rel], wv, mask=msk)
            return carry
        lax.fori_loop(0, T // 16, body, 0)

    scan(pos0_v, w0_v)
    scan(pos1_v, w1_v)
    pltpu.sync_copy(wsort_v, ws_hbm.at[pl.ds(base, SLOTS)])
    for j in range(SLOTS // GCH):
        idx = perm_v.at[pl.ds(j * GCH, GCH)]
        pltpu.async_copy(xf_hbm.at[idx], buf_v, sem).wait()
        pltpu.sync_copy(buf_v, xs_hbm.at[pl.ds(base + j * GCH, GCH)])


def _sc_dispatch(pos0, pos1, w0, w1, xf):
    mesh = plsc.VectorSubcoreMesh(core_axis_name="c", subcore_axis_name="s")
    fn = functools.partial(
        pl.kernel,
        out_type=[jax.ShapeDtypeStruct((NP, D), jnp.float32),
                  jax.ShapeDtypeStruct((NP,), jnp.float32)],
        mesh=mesh,
        scratch_types=[
            pltpu.VMEM((T,), jnp.int32),
            pltpu.VMEM((T,), jnp.int32),
            pltpu.VMEM((T,), jnp.float32),
            pltpu.VMEM((T,), jnp.float32),
            pltpu.VMEM((SLOTS,), jnp.int32),
            pltpu.VMEM((SLOTS,), jnp.float32),
            pltpu.VMEM((GCH, D), jnp.float32),
            pltpu.SemaphoreType.DMA,
        ],
        compiler_params=pltpu.CompilerParams(needs_layout_passes=False),
    )(_sc_dispatch_body)
    return fn(pos0, pos1, w0, w1, xf)


def _gmm_body(be_ref, xs_ref, w1_ref, w2_ref, ws_ref, hs_ref):
    x = xs_ref[...]
    h = _dot_t(x, w1_ref[0])
    h = h * jax.nn.sigmoid(h)
    h = _dot_t(h, w2_ref[0])
    hs_ref[...] = h * ws_ref[...]


def _gmm(be, xs, w1, w2, ws):
    grid_spec = pltpu.PrefetchScalarGridSpec(
        num_scalar_prefetch=1,
        grid=(NB,),
        in_specs=[
            pl.BlockSpec((BG, D), lambda i, s: (i, 0)),
            pl.BlockSpec((1, H, D), lambda i, s: (s[i], 0, 0)),
            pl.BlockSpec((1, D, H), lambda i, s: (s[i], 0, 0)),
            pl.BlockSpec((BG, 1), lambda i, s: (i, 0)),
        ],
        out_specs=pl.BlockSpec((BG, D), lambda i, s: (i, 0)),
    )
    return pl.pallas_call(
        _gmm_body,
        grid_spec=grid_spec,
        out_shape=jax.ShapeDtypeStruct((NP, D), jnp.float32),
    )(be, xs, w1, w2, ws)


def _sc_combine_body(pos0_hbm, pos1_hbm, hs_hbm, out_hbm,
                     p0_v, p1_v, bufa_v, bufb_v, sema, semb):
    wid = lax.axis_index("s") * 2 + lax.axis_index("c")
    tbase = wid * TOK
    pltpu.sync_copy(pos0_hbm.at[pl.ds(tbase, TOK)], p0_v)
    pltpu.sync_copy(pos1_hbm.at[pl.ds(tbase, TOK)], p1_v)
    for j in range(TOK // CCH):
        ia = p0_v.at[pl.ds(j * CCH, CCH)]
        ib = p1_v.at[pl.ds(j * CCH, CCH)]
        ca = pltpu.async_copy(hs_hbm.at[ia], bufa_v, sema)
        cb = pltpu.async_copy(hs_hbm.at[ib], bufb_v, semb)
        ca.wait()
        cb.wait()

        def addbody(i, carry):
            r = i // (D // 16)
            l = (i % (D // 16)) * 16
            a = bufa_v[r, pl.ds(l, 16)]
            bb = bufb_v[r, pl.ds(l, 16)]
            bufa_v[r, pl.ds(l, 16)] = a + bb
            return carry
        lax.fori_loop(0, CCH * (D // 16), addbody, 0)
        pltpu.sync_copy(bufa_v, out_hbm.at[pl.ds(tbase + j * CCH, CCH)])


def _sc_combine(pos0, pos1, hs):
    mesh = plsc.VectorSubcoreMesh(core_axis_name="c", subcore_axis_name="s")
    fn = functools.partial(
        pl.kernel,
        out_type=jax.ShapeDtypeStruct((T, D), jnp.float32),
        mesh=mesh,
        scratch_types=[
            pltpu.VMEM((TOK,), jnp.int32),
            pltpu.VMEM((TOK,), jnp.int32),
            pltpu.VMEM((CCH, D), jnp.float32),
            pltpu.VMEM((CCH, D), jnp.float32),
            pltpu.SemaphoreType.DMA,
            pltpu.SemaphoreType.DMA,
        ],
        compiler_params=pltpu.CompilerParams(needs_layout_passes=False),
    )(_sc_combine_body)
    return fn(pos0, pos1, hs)


def _ln_body(y_ref, g_ref, b_ref, o_ref):
    y = y_ref[...]
    mean = jnp.mean(y, axis=1, keepdims=True)
    cent = y - mean
    var = jnp.mean(cent * cent, axis=1, keepdims=True)
    o_ref[...] = cent * lax.rsqrt(var + 1e-5) * g_ref[...] + b_ref[...]


def _ln(y, gamma2, beta2):
    return pl.pallas_call(
        _ln_body,
        grid=(T // BT,),
        in_specs=[
            pl.BlockSpec((BT, D), lambda i: (i, 0)),
            pl.BlockSpec((1, D), lambda i: (0, 0)),
            pl.BlockSpec((1, D), lambda i: (0, 0)),
        ],
        out_specs=pl.BlockSpec((BT, D), lambda i: (i, 0)),
        out_shape=jax.ShapeDtypeStruct((T, D), jnp.float32),
    )(y, gamma2, beta2)


@jax.jit
def _moe_call(xf, wr, w1, w2, gamma2, beta2):
    pos0, pos1, w0c, w1c, be = _router(xf, wr)
    pos0f = pos0.reshape(T)
    pos1f = pos1.reshape(T)
    xs, ws = _sc_dispatch(pos0f, pos1f, w0c.reshape(T), w1c.reshape(T), xf)
    hs = _gmm(be.reshape(NBE), xs, w1, w2, ws.reshape(NP, 1))
    comb = _sc_combine(pos0f, pos1f, hs)
    return _ln(comb, gamma2, beta2)


def kernel(x, Wr, W1, W2, gamma, beta):
    B, S, Dm = x.shape
    xf = x.reshape(-1, Dm)
    out = _moe_call(xf, Wr, W1, W2,
                    gamma.reshape(1, Dm), beta.reshape(1, Dm))
    return out.reshape(B, S, Dm)


# speedup vs baseline: 1.0768x; 1.0768x over previous
import functools

import jax
import jax.numpy as jnp
from jax import lax
from jax.experimental import pallas as pl
from jax.experimental.pallas import tpu as pltpu
from jax.experimental.pallas import tpu_sc as plsc

D = 1024
E = 8
H = 512
T = 2048
BT = 256
NBLK = T // BT
BG = 128
NP = 4096 + E * BG
NB = NP // BG
NBE = 64
NW = 32
SLOTS = NP // NW
GCH = 40
NG = SLOTS // GCH
TOK = T // NW
CCH = 16
NC = TOK // CCH


def _dot_t(a, b):
    return lax.dot_general(a, b, (((1,), (1,)), ((), ())),
                           preferred_element_type=jnp.float32)


def _router_body(x_ref, wr_ref, pos0_ref, pos1_ref, w0_ref, w1_ref, be_ref,
                 acc_ref, base_ref, poffs_ref):
    p = pl.program_id(0)
    b = pl.program_id(1)
    x = x_ref[...]
    logits = _dot_t(x, wr_ref[...])
    iota_e = lax.broadcasted_iota(jnp.int32, (BT, E), 1)
    m0 = jnp.max(logits, axis=1, keepdims=True)
    a0 = jnp.min(jnp.where(logits >= m0, iota_e, E), axis=1, keepdims=True)
    oh0 = iota_e == a0
    masked = jnp.where(oh0, -jnp.inf, logits)
    m1 = jnp.max(masked, axis=1, keepdims=True)
    a1 = jnp.min(jnp.where(masked >= m1, iota_e, E), axis=1, keepdims=True)
    oh1 = iota_e == a1
    oh0f = oh0.astype(jnp.float32)
    oh1f = oh1.astype(jnp.float32)
    c01 = oh0f + oh1f

    @pl.when(p == 0)
    def _():
        @pl.when(b == 0)
        def _():
            acc_ref[...] = jnp.zeros((1, E), jnp.float32)
        base_ref[pl.ds(b, 1), :] = acc_ref[...]
        acc_ref[...] = acc_ref[...] + jnp.sum(c01, axis=0, keepdims=True)

    @pl.when((p == 1) & (b == 0))
    def _():
        cnti = acc_ref[...].astype(jnp.int32)
        padded = ((cnti + (BG - 1)) // BG) * BG
        paddedf = padded.astype(jnp.float32)
        r8 = lax.broadcasted_iota(jnp.int32, (E, E), 0)
        c8 = lax.broadcasted_iota(jnp.int32, (E, E), 1)
        ltri = (r8 < c8).astype(jnp.float32)
        poffs = jnp.dot(paddedf, ltri, preferred_element_type=jnp.float32)
        poffs_ref[...] = poffs
        cbf = (poffs + paddedf) * (1.0 / BG)
        bi = lax.broadcasted_iota(jnp.int32, (NBE, E), 0).astype(jnp.float32)
        be = jnp.sum((bi >= cbf[0:1, :]).astype(jnp.float32),
                     axis=1, keepdims=True)
        be_ref[...] = jnp.minimum(be, float(E - 1)).astype(jnp.int32)

    @pl.when(p == 1)
    def _():
        rr = lax.broadcasted_iota(jnp.int32, (BT, BT), 0)
        cc = lax.broadcasted_iota(jnp.int32, (BT, BT), 1)
        strict = (cc < rr).astype(jnp.float32)
        excl = jnp.dot(strict, c01, preferred_element_type=jnp.float32)
        tot = base_ref[pl.ds(b, 1), :] + excl + poffs_ref[...]
        pos0_ref[...] = jnp.sum(oh0f * tot, axis=1,
                                keepdims=True).astype(jnp.int32)
        pos1_ref[...] = jnp.sum(oh1f * tot, axis=1,
                                keepdims=True).astype(jnp.int32)
        dd = jnp.exp(m1 - m0)
        ss = 1.0 + dd
        w0_ref[...] = 1.0 / ss
        w1_ref[...] = dd / ss


def _router(xf, wr):
    return pl.pallas_call(
        _router_body,
        grid=(2, NBLK),
        in_specs=[
            pl.BlockSpec((BT, D), lambda p, b: (b, 0)),
            pl.BlockSpec((E, D), lambda p, b: (0, 0)),
        ],
        out_specs=[
            pl.BlockSpec((BT, 1), lambda p, b: (b, 0)),
            pl.BlockSpec((BT, 1), lambda p, b: (b, 0)),
            pl.BlockSpec((BT, 1), lambda p, b: (b, 0)),
            pl.BlockSpec((BT, 1), lambda p, b: (b, 0)),
            pl.BlockSpec((NBE, 1), lambda p, b: (0, 0)),
        ],
        out_shape=[
            jax.ShapeDtypeStruct((T, 1), jnp.int32),
            jax.ShapeDtypeStruct((T, 1), jnp.int32),
            jax.ShapeDtypeStruct((T, 1), jnp.float32),
            jax.ShapeDtypeStruct((T, 1), jnp.float32),
            jax.ShapeDtypeStruct((NBE, 1), jnp.int32),
        ],
        scratch_shapes=[
            pltpu.VMEM((1, E), jnp.float32),
            pltpu.VMEM((NBLK, E), jnp.float32),
            pltpu.VMEM((1, E), jnp.float32),
        ],
    )(xf, wr)


def _sc_dispatch_body(pos0_hbm, pos1_hbm, w0_hbm, w1_hbm, xf_hbm,
                      xs_hbm, ws_hbm,
                      pos0_v, pos1_v, w0_v, w1_v, perm_v, wsort_v,
                      buf_v, buf2_v, sem, sem2):
    wid = lax.axis_index("s") * 2 + lax.axis_index("c")
    base = wid * SLOTS
    pltpu.sync_copy(pos0_hbm, pos0_v)
    pltpu.sync_copy(pos1_hbm, pos1_v)
    pltpu.sync_copy(w0_hbm, w0_v)
    pltpu.sync_copy(w1_hbm, w1_v)
    zi = jnp.zeros((16,), jnp.int32)
    zf = jnp.zeros((16,), jnp.float32)
    for i in range(SLOTS // 16):
        perm_v[pl.ds(i * 16, 16)] = zi
        wsort_v[pl.ds(i * 16, 16)] = zf
    lane = lax.iota(jnp.int32, 16)

    def scan(pos_ref, w_ref):
        def body(c, carry):
            off = c * 16
            pv = pos_ref[pl.ds(off, 16)]
            wv = w_ref[pl.ds(off, 16)]
            tv = lane + off
            msk = (pv >= base) & (pv < base + SLOTS)
            rel = jnp.where(msk, pv - base, 0)
            plsc.store_scatter(perm_v, [rel], tv, mask=msk)
            plsc.store_scatter(wsort_v, [rel], wv, mask=msk)
            return carry
        lax.fori_loop(0, T // 16, body, 0)

    scan(pos0_v, w0_v)
    scan(pos1_v, w1_v)
    pltpu.sync_copy(wsort_v, ws_hbm.at[pl.ds(base, SLOTS)])
    bufs = (buf_v, buf2_v)
    sems = (sem, sem2)
    copies = [None] * NG
    copies[0] = pltpu.async_copy(
        xf_hbm.at[perm_v.at[pl.ds(0, GCH)]], bufs[0], sems[0])
    for j in range(NG):
        copies[j].wait()
        if j + 1 < NG:
            copies[j + 1] = pltpu.async_copy(
                xf_hbm.at[perm_v.at[pl.ds((j + 1) * GCH, GCH)]],
                bufs[(j + 1) % 2], sems[(j + 1) % 2])
        pltpu.sync_copy(bufs[j % 2], xs_hbm.at[pl.ds(base + j * GCH, GCH)])


def _sc_dispatch(pos0, pos1, w0, w1, xf):
    mesh = plsc.VectorSubcoreMesh(core_axis_name="c", subcore_axis_name="s")
    fn = functools.partial(
        pl.kernel,
        out_type=[jax.ShapeDtypeStruct((NP, D), jnp.float32),
                  jax.ShapeDtypeStruct((NP,), jnp.float32)],
        mesh=mesh,
        scratch_types=[
            pltpu.VMEM((T,), jnp.int32),
            pltpu.VMEM((T,), jnp.int32),
            pltpu.VMEM((T,), jnp.float32),
            pltpu.VMEM((T,), jnp.float32),
            pltpu.VMEM((SLOTS,), jnp.int32),
            pltpu.VMEM((SLOTS,), jnp.float32),
            pltpu.VMEM((GCH, D), jnp.float32),
            pltpu.VMEM((GCH, D), jnp.float32),
            pltpu.SemaphoreType.DMA,
            pltpu.SemaphoreType.DMA,
        ],
        compiler_params=pltpu.CompilerParams(needs_layout_passes=False),
    )(_sc_dispatch_body)
    return fn(pos0, pos1, w0, w1, xf)


def _gmm_body(be_ref, xs_ref, w1_ref, w2_ref, ws_ref, hs_ref):
    x = xs_ref[...]
    h = _dot_t(x, w1_ref[0])
    h = h * jax.nn.sigmoid(h)
    h = _dot_t(h, w2_ref[0])
    hs_ref[...] = h * ws_ref[...]


def _gmm(be, xs, w1, w2, ws):
    grid_spec = pltpu.PrefetchScalarGridSpec(
        num_scalar_prefetch=1,
        grid=(NB,),
        in_specs=[
            pl.BlockSpec((BG, D), lambda i, s: (i, 0)),
            pl.BlockSpec((1, H, D), lambda i, s: (s[i], 0, 0)),
            pl.BlockSpec((1, D, H), lambda i, s: (s[i], 0, 0)),
            pl.BlockSpec((BG, 1), lambda i, s: (i, 0)),
        ],
        out_specs=pl.BlockSpec((BG, D), lambda i, s: (i, 0)),
    )
    return pl.pallas_call(
        _gmm_body,
        grid_spec=grid_spec,
        out_shape=jax.ShapeDtypeStruct((NP, D), jnp.float32),
    )(be, xs, w1, w2, ws)


def _sc_combine_body(pos0_hbm, pos1_hbm, hs_hbm, out_hbm,
                     p0_v, p1_v, ba0_v, bb0_v, ba1_v, bb1_v,
                     sa0, sb0, sa1, sb1):
    wid = lax.axis_index("s") * 2 + lax.axis_index("c")
    tbase = wid * TOK
    pltpu.sync_copy(pos0_hbm.at[pl.ds(tbase, TOK)], p0_v)
    pltpu.sync_copy(pos1_hbm.at[pl.ds(tbase, TOK)], p1_v)
    bufa = (ba0_v, ba1_v)
    bufb = (bb0_v, bb1_v)
    sema = (sa0, sa1)
    semb = (sb0, sb1)

    def fire(j):
        ia = p0_v.at[pl.ds(j * CCH, CCH)]
        ib = p1_v.at[pl.ds(j * CCH, CCH)]
        return (pltpu.async_copy(hs_hbm.at[ia], bufa[j % 2], sema[j % 2]),
                pltpu.async_copy(hs_hbm.at[ib], bufb[j % 2], semb[j % 2]))

    cps = [None] * NC
    cps[0] = fire(0)
    for j in range(NC):
        cps[j][0].wait()
        cps[j][1].wait()
        if j + 1 < NC:
            cps[j + 1] = fire(j + 1)
        ba = bufa[j % 2]
        bb = bufb[j % 2]

        def addrow(r, carry):
            for u in range(D // 16):
                ba[r, pl.ds(u * 16, 16)] = (ba[r, pl.ds(u * 16, 16)]
                                            + bb[r, pl.ds(u * 16, 16)])
            return carry
        lax.fori_loop(0, CCH, addrow, 0)
        pltpu.sync_copy(ba, out_hbm.at[pl.ds(tbase + j * CCH, CCH)])


def _sc_combine(pos0, pos1, hs):
    mesh = plsc.VectorSubcoreMesh(core_axis_name="c", subcore_axis_name="s")
    fn = functools.partial(
        pl.kernel,
        out_type=jax.ShapeDtypeStruct((T, D), jnp.float32),
        mesh=mesh,
        scratch_types=[
            pltpu.VMEM((TOK,), jnp.int32),
            pltpu.VMEM((TOK,), jnp.int32),
            pltpu.VMEM((CCH, D), jnp.float32),
            pltpu.VMEM((CCH, D), jnp.float32),
            pltpu.VMEM((CCH, D), jnp.float32),
            pltpu.VMEM((CCH, D), jnp.float32),
            pltpu.SemaphoreType.DMA,
            pltpu.SemaphoreType.DMA,
            pltpu.SemaphoreType.DMA,
            pltpu.SemaphoreType.DMA,
        ],
        compiler_params=pltpu.CompilerParams(needs_layout_passes=False),
    )(_sc_combine_body)
    return fn(pos0, pos1, hs)


def _ln_body(y_ref, g_ref, b_ref, o_ref):
    y = y_ref[...]
    mean = jnp.mean(y, axis=1, keepdims=True)
    cent = y - mean
    var = jnp.mean(cent * cent, axis=1, keepdims=True)
    o_ref[...] = cent * lax.rsqrt(var + 1e-5) * g_ref[...] + b_ref[...]


def _ln(y, gamma2, beta2):
    return pl.pallas_call(
        _ln_body,
        grid=(T // BT,),
        in_specs=[
            pl.BlockSpec((BT, D), lambda i: (i, 0)),
            pl.BlockSpec((1, D), lambda i: (0, 0)),
            pl.BlockSpec((1, D), lambda i: (0, 0)),
        ],
        out_specs=pl.BlockSpec((BT, D), lambda i: (i, 0)),
        out_shape=jax.ShapeDtypeStruct((T, D), jnp.float32),
    )(y, gamma2, beta2)


@jax.jit
def _moe_call(xf, wr, w1, w2, gamma2, beta2):
    pos0, pos1, w0c, w1c, be = _router(xf, wr)
    pos0f = pos0.reshape(T)
    pos1f = pos1.reshape(T)
    xs, ws = _sc_dispatch(pos0f, pos1f, w0c.reshape(T), w1c.reshape(T), xf)
    hs = _gmm(be.reshape(NBE), xs, w1, w2, ws.reshape(NP, 1))
    comb = _sc_combine(pos0f, pos1f, hs)
    return _ln(comb, gamma2, beta2)


def kernel(x, Wr, W1, W2, gamma, beta):
    B, S, Dm = x.shape
    xf = x.reshape(-1, Dm)
    out = _moe_call(xf, Wr, W1, W2,
                    gamma.reshape(1, Dm), beta.reshape(1, Dm))
    return out.reshape(B, S, Dm)


# speedup vs baseline: 1.1235x; 1.0435x over previous
import functools

import jax
import jax.numpy as jnp
from jax import lax
from jax.experimental import pallas as pl
from jax.experimental.pallas import tpu as pltpu
from jax.experimental.pallas import tpu_sc as plsc

D = 1024
E = 8
H = 512
T = 2048
BT = 256
NBLK = T // BT
BG = 128
NP = 4096 + E * BG
NB = NP // BG
NBE = 64
NW = 32
SLOTS = NP // NW
GCH = 32
NG = SLOTS // GCH
NBUF = 3
TOK = T // NW
CCH = 16
NC = TOK // CCH


def _dot_t(a, b):
    return lax.dot_general(a, b, (((1,), (1,)), ((), ())),
                           preferred_element_type=jnp.float32)


def _router_body(x_ref, wr_ref, pos0_ref, pos1_ref, w0_ref, w1_ref, be_ref):
    x = x_ref[...]
    logits = _dot_t(x, wr_ref[...])
    iota_e = lax.broadcasted_iota(jnp.int32, (T, E), 1)
    m0 = jnp.max(logits, axis=1, keepdims=True)
    a0 = jnp.min(jnp.where(logits >= m0, iota_e, E), axis=1, keepdims=True)
    oh0 = iota_e == a0
    masked = jnp.where(oh0, -jnp.inf, logits)
    m1 = jnp.max(masked, axis=1, keepdims=True)
    a1 = jnp.min(jnp.where(masked >= m1, iota_e, E), axis=1, keepdims=True)
    oh1 = iota_e == a1
    oh0f = oh0.astype(jnp.float32)
    oh1f = oh1.astype(jnp.float32)
    c01 = oh0f + oh1f

    rr = lax.broadcasted_iota(jnp.int32, (T, T), 0)
    cc = lax.broadcasted_iota(jnp.int32, (T, T), 1)
    strict = (cc < rr).astype(jnp.float32)
    excl = jnp.dot(strict, c01, preferred_element_type=jnp.float32)

    cnt = jnp.sum(c01, axis=0, keepdims=True)
    cnti = cnt.astype(jnp.int32)
    padded = ((cnti + (BG - 1)) // BG) * BG
    paddedf = padded.astype(jnp.float32)
    r8 = lax.broadcasted_iota(jnp.int32, (E, E), 0)
    c8 = lax.broadcasted_iota(jnp.int32, (E, E), 1)
    ltri = (r8 < c8).astype(jnp.float32)
    poffs = jnp.dot(paddedf, ltri, preferred_element_type=jnp.float32)

    tot = excl + poffs
    pos0_ref[...] = jnp.sum(oh0f * tot, axis=1,
                            keepdims=True).astype(jnp.int32)
    pos1_ref[...] = jnp.sum(oh1f * tot, axis=1,
                            keepdims=True).astype(jnp.int32)
    dd = jnp.exp(m1 - m0)
    ss = 1.0 + dd
    w0_ref[...] = 1.0 / ss
    w1_ref[...] = dd / ss

    cbf = (poffs + paddedf) * (1.0 / BG)
    bi = lax.broadcasted_iota(jnp.int32, (NBE, E), 0).astype(jnp.float32)
    be = jnp.sum((bi >= cbf[0:1, :]).astype(jnp.float32),
                 axis=1, keepdims=True)
    be_ref[...] = jnp.minimum(be, float(E - 1)).astype(jnp.int32)


def _router(xf, wr):
    return pl.pallas_call(
        _router_body,
        out_shape=[
            jax.ShapeDtypeStruct((T, 1), jnp.int32),
            jax.ShapeDtypeStruct((T, 1), jnp.int32),
            jax.ShapeDtypeStruct((T, 1), jnp.float32),
            jax.ShapeDtypeStruct((T, 1), jnp.float32),
            jax.ShapeDtypeStruct((NBE, 1), jnp.int32),
        ],
    )(xf, wr)


def _sc_dispatch_body(pos0_hbm, pos1_hbm, w0_hbm, w1_hbm, xf_hbm,
                      xs_hbm, ws_hbm,
                      pos0_v, pos1_v, w0_v, w1_v, perm_v, wsort_v,
                      buf_v, buf2_v, buf3_v, sem, sem2, sem3):
    wid = lax.axis_index("s") * 2 + lax.axis_index("c")
    base = wid * SLOTS
    pltpu.sync_copy(pos0_hbm, pos0_v)
    pltpu.sync_copy(pos1_hbm, pos1_v)
    pltpu.sync_copy(w0_hbm, w0_v)
    pltpu.sync_copy(w1_hbm, w1_v)
    zi = jnp.zeros((16,), jnp.int32)
    zf = jnp.zeros((16,), jnp.float32)
    for i in range(SLOTS // 16):
        perm_v[pl.ds(i * 16, 16)] = zi
        wsort_v[pl.ds(i * 16, 16)] = zf
    lane = lax.iota(jnp.int32, 16)

    def scan(pos_ref, w_ref):
        def body(c, carry):
            off = c * 16
            pv = pos_ref[pl.ds(off, 16)]
            wv = w_ref[pl.ds(off, 16)]
            tv = lane + off
            msk = (pv >= base) & (pv < base + SLOTS)
            rel = jnp.where(msk, pv - base, 0)
            plsc.store_scatter(perm_v, [rel], tv, mask=msk)
            plsc.store_scatter(wsort_v, [rel], wv, mask=msk)
            return carry
        lax.fori_loop(0, T // 16, body, 0)

    scan(pos0_v, w0_v)
    scan(pos1_v, w1_v)
    pltpu.sync_copy(wsort_v, ws_hbm.at[pl.ds(base, SLOTS)])
    bufs = (buf_v, buf2_v, buf3_v)
    sems = (sem, sem2, sem3)

    def fire(j):
        return pltpu.async_copy(
            xf_hbm.at[perm_v.at[pl.ds(j * GCH, GCH)]],
            bufs[j % NBUF], sems[j % NBUF])

    copies = [None] * NG
    for j in range(NBUF):
        copies[j] = fire(j)
    for j in range(NG):
        copies[j].wait()
        pltpu.sync_copy(bufs[j % NBUF], xs_hbm.at[pl.ds(base + j * GCH, GCH)])
        if j + NBUF < NG:
            copies[j + NBUF] = fire(j + NBUF)


def _sc_dispatch(pos0, pos1, w0, w1, xf):
    mesh = plsc.VectorSubcoreMesh(core_axis_name="c", subcore_axis_name="s")
    fn = functools.partial(
        pl.kernel,
        out_type=[jax.ShapeDtypeStruct((NP, D), jnp.float32),
                  jax.ShapeDtypeStruct((NP,), jnp.float32)],
        mesh=mesh,
        scratch_types=[
            pltpu.VMEM((T,), jnp.int32),
            pltpu.VMEM((T,), jnp.int32),
            pltpu.VMEM((T,), jnp.float32),
            pltpu.VMEM((T,), jnp.float32),
            pltpu.VMEM((SLOTS,), jnp.int32),
            pltpu.VMEM((SLOTS,), jnp.float32),
            pltpu.VMEM((GCH, D), jnp.float32),
            pltpu.VMEM((GCH, D), jnp.float32),
            pltpu.VMEM((GCH, D), jnp.float32),
            pltpu.SemaphoreType.DMA,
            pltpu.SemaphoreType.DMA,
            pltpu.SemaphoreType.DMA,
        ],
        compiler_params=pltpu.CompilerParams(needs_layout_passes=False),
    )(_sc_dispatch_body)
    return fn(pos0, pos1, w0, w1, xf)


def _gmm_body(be_ref, xs_ref, w1_ref, w2_ref, ws_ref, hs_ref):
    i = pl.program_id(0)
    e = be_ref[i, 0]
    x = xs_ref[...]
    w1e = w1_ref[pl.ds(e, 1)][0]
    w2e = w2_ref[pl.ds(e, 1)][0]
    h = _dot_t(x, w1e)
    h = h * jax.nn.sigmoid(h)
    h = _dot_t(h, w2e)
    hs_ref[...] = h * ws_ref[...]


def _gmm(be, xs, w1, w2, ws):
    return pl.pallas_call(
        _gmm_body,
        grid=(NB,),
        in_specs=[
            pl.BlockSpec(memory_space=pltpu.SMEM),
            pl.BlockSpec((BG, D), lambda i: (i, 0)),
            pl.BlockSpec((E, H, D), lambda i: (0, 0, 0)),
            pl.BlockSpec((E, D, H), lambda i: (0, 0, 0)),
            pl.BlockSpec((BG, 1), lambda i: (i, 0)),
        ],
        out_specs=pl.BlockSpec((BG, D), lambda i: (i, 0)),
        out_shape=jax.ShapeDtypeStruct((NP, D), jnp.float32),
    )(be, xs, w1, w2, ws)


def _sc_combine_body(pos0_hbm, pos1_hbm, hs_hbm, out_hbm,
                     p0_v, p1_v, ba0_v, bb0_v, ba1_v, bb1_v, ba2_v, bb2_v,
                     sa0, sb0, sa1, sb1, sa2, sb2):
    wid = lax.axis_index("s") * 2 + lax.axis_index("c")
    tbase = wid * TOK
    pltpu.sync_copy(pos0_hbm.at[pl.ds(tbase, TOK)], p0_v)
    pltpu.sync_copy(pos1_hbm.at[pl.ds(tbase, TOK)], p1_v)
    bufa = (ba0_v, ba1_v, ba2_v)
    bufb = (bb0_v, bb1_v, bb2_v)
    sema = (sa0, sa1, sa2)
    semb = (sb0, sb1, sb2)

    def fire(j):
        ia = p0_v.at[pl.ds(j * CCH, CCH)]
        ib = p1_v.at[pl.ds(j * CCH, CCH)]
        return (pltpu.async_copy(hs_hbm.at[ia], bufa[j % 3], sema[j % 3]),
                pltpu.async_copy(hs_hbm.at[ib], bufb[j % 3], semb[j % 3]))

    cps = [None] * NC
    cps[0] = fire(0)
    cps[1] = fire(1)
    cps[2] = fire(2)
    for j in range(NC):
        cps[j][0].wait()
        cps[j][1].wait()
        ba = bufa[j % 3]
        bb = bufb[j % 3]

        def addrow(r, carry):
            for u in range(D // 16):
                ba[r, pl.ds(u * 16, 16)] = (ba[r, pl.ds(u * 16, 16)]
                                            + bb[r, pl.ds(u * 16, 16)])
            return carry
        lax.fori_loop(0, CCH, addrow, 0)
        pltpu.sync_copy(ba, out_hbm.at[pl.ds(tbase + j * CCH, CCH)])
        if j + 3 < NC:
            cps[j + 3] = fire(j + 3)


def _sc_combine(pos0, pos1, hs):
    mesh = plsc.VectorSubcoreMesh(core_axis_name="c", subcore_axis_name="s")
    fn = functools.partial(
        pl.kernel,
        out_type=jax.ShapeDtypeStruct((T, D), jnp.float32),
        mesh=mesh,
        scratch_types=[
            pltpu.VMEM((TOK,), jnp.int32),
            pltpu.VMEM((TOK,), jnp.int32),
            pltpu.VMEM((CCH, D), jnp.float32),
            pltpu.VMEM((CCH, D), jnp.float32),
            pltpu.VMEM((CCH, D), jnp.float32),
            pltpu.VMEM((CCH, D), jnp.float32),
            pltpu.VMEM((CCH, D), jnp.float32),
            pltpu.VMEM((CCH, D), jnp.float32),
            pltpu.SemaphoreType.DMA,
            pltpu.SemaphoreType.DMA,
            pltpu.SemaphoreType.DMA,
            pltpu.SemaphoreType.DMA,
            pltpu.SemaphoreType.DMA,
            pltpu.SemaphoreType.DMA,
        ],
        compiler_params=pltpu.CompilerParams(needs_layout_passes=False),
    )(_sc_combine_body)
    return fn(pos0, pos1, hs)


def _ln_body(y_ref, g_ref, b_ref, o_ref):
    y = y_ref[...]
    mean = jnp.mean(y, axis=1, keepdims=True)
    cent = y - mean
    var = jnp.mean(cent * cent, axis=1, keepdims=True)
    o_ref[...] = cent * lax.rsqrt(var + 1e-5) * g_ref[...] + b_ref[...]


def _ln(y, gamma2, beta2):
    return pl.pallas_call(
        _ln_body,
        grid=(T // BT,),
        in_specs=[
            pl.BlockSpec((BT, D), lambda i: (i, 0)),
            pl.BlockSpec((1, D), lambda i: (0, 0)),
            pl.BlockSpec((1, D), lambda i: (0, 0)),
        ],
        out_specs=pl.BlockSpec((BT, D), lambda i: (i, 0)),
        out_shape=jax.ShapeDtypeStruct((T, D), jnp.float32),
    )(y, gamma2, beta2)


@jax.jit
def _moe_call(xf, wr, w1, w2, gamma2, beta2):
    pos0, pos1, w0c, w1c, be = _router(xf, wr)
    pos0f = pos0.reshape(T)
    pos1f = pos1.reshape(T)
    xs, ws = _sc_dispatch(pos0f, pos1f, w0c.reshape(T), w1c.reshape(T), xf)
    hs = _gmm(be, xs, w1, w2, ws.reshape(NP, 1))
    comb = _sc_combine(pos0f, pos1f, hs)
    return _ln(comb, gamma2, beta2)


def kernel(x, Wr, W1, W2, gamma, beta):
    B, S, Dm = x.shape
    xf = x.reshape(-1, Dm)
    out = _moe_call(xf, Wr, W1, W2,
                    gamma.reshape(1, Dm), beta.reshape(1, Dm))
    return out.reshape(B, S, Dm)


# speedup vs baseline: 1.1369x; 1.0119x over previous
import functools

import jax
import jax.numpy as jnp
from jax import lax
from jax.experimental import pallas as pl
from jax.experimental.pallas import tpu as pltpu
from jax.experimental.pallas import tpu_sc as plsc

D = 1024
E = 8
H = 512
T = 2048
BT = 256
NBLK = T // BT
BG = 128
NP = 4096 + E * BG
NB = NP // BG
NBE = 64
NW = 32
SLOTS = NP // NW
GCH = 32
NG = SLOTS // GCH
NBUF = 3
TOK = T // NW
CCH = 16
NC = TOK // CCH


def _dot_t(a, b):
    return lax.dot_general(a, b, (((1,), (1,)), ((), ())),
                           preferred_element_type=jnp.float32)


def _router_body(x_ref, wr_ref, pos0_ref, pos1_ref, w0_ref, w1_ref, be_ref):
    x = x_ref[...]
    logits = _dot_t(x, wr_ref[...])
    iota_e = lax.broadcasted_iota(jnp.int32, (T, E), 1)
    m0 = jnp.max(logits, axis=1, keepdims=True)
    a0 = jnp.min(jnp.where(logits >= m0, iota_e, E), axis=1, keepdims=True)
    oh0 = iota_e == a0
    masked = jnp.where(oh0, -jnp.inf, logits)
    m1 = jnp.max(masked, axis=1, keepdims=True)
    a1 = jnp.min(jnp.where(masked >= m1, iota_e, E), axis=1, keepdims=True)
    oh1 = iota_e == a1
    oh0f = oh0.astype(jnp.float32)
    oh1f = oh1.astype(jnp.float32)
    c01 = oh0f + oh1f

    rr = lax.broadcasted_iota(jnp.int32, (T, T), 0)
    cc = lax.broadcasted_iota(jnp.int32, (T, T), 1)
    strict = (cc < rr).astype(jnp.float32)
    excl = jnp.dot(strict, c01, preferred_element_type=jnp.float32)

    cnt = jnp.sum(c01, axis=0, keepdims=True)
    cnti = cnt.astype(jnp.int32)
    padded = ((cnti + (BG - 1)) // BG) * BG
    paddedf = padded.astype(jnp.float32)
    r8 = lax.broadcasted_iota(jnp.int32, (E, E), 0)
    c8 = lax.broadcasted_iota(jnp.int32, (E, E), 1)
    ltri = (r8 < c8).astype(jnp.float32)
    poffs = jnp.dot(paddedf, ltri, preferred_element_type=jnp.float32)

    tot = excl + poffs
    pos0_ref[...] = jnp.sum(oh0f * tot, axis=1,
                            keepdims=True).astype(jnp.int32)
    pos1_ref[...] = jnp.sum(oh1f * tot, axis=1,
                            keepdims=True).astype(jnp.int32)
    dd = jnp.exp(m1 - m0)
    ss = 1.0 + dd
    w0_ref[...] = 1.0 / ss
    w1_ref[...] = dd / ss

    cbf = (poffs + paddedf) * (1.0 / BG)
    bi = lax.broadcasted_iota(jnp.int32, (NBE, E), 0).astype(jnp.float32)
    be = jnp.sum((bi >= cbf[0:1, :]).astype(jnp.float32),
                 axis=1, keepdims=True)
    be_ref[...] = jnp.minimum(be, float(E - 1)).astype(jnp.int32)


def _router(xf, wr):
    return pl.pallas_call(
        _router_body,
        out_shape=[
            jax.ShapeDtypeStruct((T, 1), jnp.int32),
            jax.ShapeDtypeStruct((T, 1), jnp.int32),
            jax.ShapeDtypeStruct((T, 1), jnp.float32),
            jax.ShapeDtypeStruct((T, 1), jnp.float32),
            jax.ShapeDtypeStruct((NBE, 1), jnp.int32),
        ],
    )(xf, wr)


def _sc_dispatch_body(pos0_hbm, pos1_hbm, w0_hbm, w1_hbm, xf_hbm,
                      xs_hbm, ws_hbm,
                      pos0_v, pos1_v, w0_v, w1_v, perm_v, wsort_v,
                      idx_v, buf_v, buf2_v, buf3_v, shri_v, shrf_v,
                      sem, sem2, sem3):
    sid = lax.axis_index("s")
    wid = sid * 2 + lax.axis_index("c")
    base = wid * SLOTS
    @pl.when(sid == 0)
    def _():
        pltpu.sync_copy(pos0_hbm, shri_v.at[0])
        pltpu.sync_copy(pos1_hbm, shri_v.at[1])
        pltpu.sync_copy(w0_hbm, shrf_v.at[0])
        pltpu.sync_copy(w1_hbm, shrf_v.at[1])
    plsc.subcore_barrier()
    pltpu.sync_copy(shri_v.at[0], pos0_v)
    pltpu.sync_copy(shri_v.at[1], pos1_v)
    pltpu.sync_copy(shrf_v.at[0], w0_v)
    pltpu.sync_copy(shrf_v.at[1], w1_v)
    zi = jnp.zeros((16,), jnp.int32)
    zf = jnp.zeros((16,), jnp.float32)
    for i in range(SLOTS // 16):
        perm_v[pl.ds(i * 16, 16)] = zi
        wsort_v[pl.ds(i * 16, 16)] = zf
    lane = lax.iota(jnp.int32, 16)

    def scan(pos_ref, w_ref):
        def body(c, carry):
            off = c * 16
            pv = pos_ref[pl.ds(off, 16)]
            wv = w_ref[pl.ds(off, 16)]
            tv = lane + off
            msk = (pv >= base) & (pv < base + SLOTS)
            rel = jnp.where(msk, pv - base, 0)
            plsc.store_scatter(perm_v, [rel], tv, mask=msk)
            plsc.store_scatter(wsort_v, [rel], wv, mask=msk)
            return carry
        lax.fori_loop(0, T // 16, body, 0)

    scan(pos0_v, w0_v)
    scan(pos1_v, w1_v)
    pltpu.sync_copy(wsort_v, ws_hbm.at[pl.ds(base, SLOTS)])
    bufs = (buf_v, buf2_v, buf3_v)
    sems = (sem, sem2, sem3)

    def fire(j):
        for u in range(GCH // 16):
            idx_v[j % NBUF, pl.ds(u * 16, 16)] = (
                perm_v[pl.ds(j * GCH + u * 16, 16)])
        return pltpu.async_copy(
            xf_hbm.at[idx_v.at[j % NBUF]],
            bufs[j % NBUF], sems[j % NBUF])

    copies = [None] * NG
    for j in range(NBUF):
        copies[j] = fire(j)
    for j in range(NG):
        copies[j].wait()
        pltpu.sync_copy(bufs[j % NBUF], xs_hbm.at[pl.ds(base + j * GCH, GCH)])
        if j + NBUF < NG:
            copies[j + NBUF] = fire(j + NBUF)


def _sc_dispatch(pos0, pos1, w0, w1, xf):
    mesh = plsc.VectorSubcoreMesh(core_axis_name="c", subcore_axis_name="s")
    fn = functools.partial(
        pl.kernel,
        out_type=[jax.ShapeDtypeStruct((NP, D), jnp.float32),
                  jax.ShapeDtypeStruct((NP,), jnp.float32)],
        mesh=mesh,
        scratch_types=[
            pltpu.VMEM((T,), jnp.int32),
            pltpu.VMEM((T,), jnp.int32),
            pltpu.VMEM((T,), jnp.float32),
            pltpu.VMEM((T,), jnp.float32),
            pltpu.VMEM((SLOTS,), jnp.int32),
            pltpu.VMEM((SLOTS,), jnp.float32),
            pltpu.VMEM((NBUF, GCH), jnp.int32),
            pltpu.VMEM((GCH, D), jnp.float32),
            pltpu.VMEM((GCH, D), jnp.float32),
            pltpu.VMEM((GCH, D), jnp.float32),
            pltpu.VMEM_SHARED((2, T), jnp.int32),
            pltpu.VMEM_SHARED((2, T), jnp.float32),
            pltpu.SemaphoreType.DMA,
            pltpu.SemaphoreType.DMA,
            pltpu.SemaphoreType.DMA,
        ],
        compiler_params=pltpu.CompilerParams(needs_layout_passes=False),
    )(_sc_dispatch_body)
    return fn(pos0, pos1, w0, w1, xf)


def _gmm_body(be_ref, xs_ref, w1_ref, w2_ref, ws_ref, hs_ref):
    i = pl.program_id(0)
    e = be_ref[i, 0]
    x = xs_ref[...]
    w1e = w1_ref[pl.ds(e, 1)][0]
    w2e = w2_ref[pl.ds(e, 1)][0]
    h = _dot_t(x, w1e)
    h = h * jax.nn.sigmoid(h)
    h = _dot_t(h, w2e)
    hs_ref[...] = h * ws_ref[...]


def _gmm(be, xs, w1, w2, ws):
    return pl.pallas_call(
        _gmm_body,
        grid=(NB,),
        in_specs=[
            pl.BlockSpec(memory_space=pltpu.SMEM),
            pl.BlockSpec((BG, D), lambda i: (i, 0)),
            pl.BlockSpec((E, H, D), lambda i: (0, 0, 0)),
            pl.BlockSpec((E, D, H), lambda i: (0, 0, 0)),
            pl.BlockSpec((BG, 1), lambda i: (i, 0)),
        ],
        out_specs=pl.BlockSpec((BG, D), lambda i: (i, 0)),
        out_shape=jax.ShapeDtypeStruct((NP, D), jnp.float32),
    )(be, xs, w1, w2, ws)


def _sc_combine_body(pos0_hbm, pos1_hbm, hs_hbm, out_hbm,
                     p0_v, p1_v, ba0_v, bb0_v, ba1_v, bb1_v, ba2_v, bb2_v,
                     sa0, sb0, sa1, sb1, sa2, sb2):
    wid = lax.axis_index("s") * 2 + lax.axis_index("c")
    tbase = wid * TOK
    pltpu.sync_copy(pos0_hbm.at[pl.ds(tbase, TOK)], p0_v)
    pltpu.sync_copy(pos1_hbm.at[pl.ds(tbase, TOK)], p1_v)
    bufa = (ba0_v, ba1_v, ba2_v)
    bufb = (bb0_v, bb1_v, bb2_v)
    sema = (sa0, sa1, sa2)
    semb = (sb0, sb1, sb2)

    def fire(j):
        ia = p0_v.at[pl.ds(j * CCH, CCH)]
        ib = p1_v.at[pl.ds(j * CCH, CCH)]
        return (pltpu.async_copy(hs_hbm.at[ia], bufa[j % 3], sema[j % 3]),
                pltpu.async_copy(hs_hbm.at[ib], bufb[j % 3], semb[j % 3]))

    cps = [None] * NC
    cps[0] = fire(0)
    cps[1] = fire(1)
    cps[2] = fire(2)
    for j in range(NC):
        cps[j][0].wait()
        cps[j][1].wait()
        ba = bufa[j % 3]
        bb = bufb[j % 3]

        def addrow(r, carry):
            for u in range(D // 16):
                ba[r, pl.ds(u * 16, 16)] = (ba[r, pl.ds(u * 16, 16)]
                                            + bb[r, pl.ds(u * 16, 16)])
            return carry
        lax.fori_loop(0, CCH, addrow, 0)
        pltpu.sync_copy(ba, out_hbm.at[pl.ds(tbase + j * CCH, CCH)])
        if j + 3 < NC:
            cps[j + 3] = fire(j + 3)


def _sc_combine(pos0, pos1, hs):
    mesh = plsc.VectorSubcoreMesh(core_axis_name="c", subcore_axis_name="s")
    fn = functools.partial(
        pl.kernel,
        out_type=jax.ShapeDtypeStruct((T, D), jnp.float32),
        mesh=mesh,
        scratch_types=[
            pltpu.VMEM((TOK,), jnp.int32),
            pltpu.VMEM((TOK,), jnp.int32),
            pltpu.VMEM((CCH, D), jnp.float32),
            pltpu.VMEM((CCH, D), jnp.float32),
            pltpu.VMEM((CCH, D), jnp.float32),
            pltpu.VMEM((CCH, D), jnp.float32),
            pltpu.VMEM((CCH, D), jnp.float32),
            pltpu.VMEM((CCH, D), jnp.float32),
            pltpu.SemaphoreType.DMA,
            pltpu.SemaphoreType.DMA,
            pltpu.SemaphoreType.DMA,
            pltpu.SemaphoreType.DMA,
            pltpu.SemaphoreType.DMA,
            pltpu.SemaphoreType.DMA,
        ],
        compiler_params=pltpu.CompilerParams(needs_layout_passes=False),
    )(_sc_combine_body)
    return fn(pos0, pos1, hs)


def _ln_body(y_ref, g_ref, b_ref, o_ref):
    y = y_ref[...]
    mean = jnp.mean(y, axis=1, keepdims=True)
    cent = y - mean
    var = jnp.mean(cent * cent, axis=1, keepdims=True)
    o_ref[...] = cent * lax.rsqrt(var + 1e-5) * g_ref[...] + b_ref[...]


def _ln(y, gamma2, beta2):
    return pl.pallas_call(
        _ln_body,
        grid=(T // BT,),
        in_specs=[
            pl.BlockSpec((BT, D), lambda i: (i, 0)),
            pl.BlockSpec((1, D), lambda i: (0, 0)),
            pl.BlockSpec((1, D), lambda i: (0, 0)),
        ],
        out_specs=pl.BlockSpec((BT, D), lambda i: (i, 0)),
        out_shape=jax.ShapeDtypeStruct((T, D), jnp.float32),
    )(y, gamma2, beta2)


@jax.jit
def _moe_call(xf, wr, w1, w2, gamma2, beta2):
    pos0, pos1, w0c, w1c, be = _router(xf, wr)
    pos0f = pos0.reshape(T)
    pos1f = pos1.reshape(T)
    xs, ws = _sc_dispatch(pos0f, pos1f, w0c.reshape(T), w1c.reshape(T), xf)
    hs = _gmm(be, xs, w1, w2, ws.reshape(NP, 1))
    comb = _sc_combine(pos0f, pos1f, hs)
    return _ln(comb, gamma2, beta2)


def kernel(x, Wr, W1, W2, gamma, beta):
    B, S, Dm = x.shape
    xf = x.reshape(-1, Dm)
    out = _moe_call(xf, Wr, W1, W2,
                    gamma.reshape(1, Dm), beta.reshape(1, Dm))
    return out.reshape(B, S, Dm)
